# baseline (speedup 1.0000x reference)
import functools

import jax
import jax.numpy as jnp
from jax import lax
from jax.experimental import pallas as pl
from jax.experimental.pallas import tpu as pltpu

B, S, D = 4, 8192, 1024
NK = 1638
NK_PAD = 1664
W = 8
FLAT = B * NK


def _cumsum_lanes(a):
    n = a.shape[-1]
    s = 1
    while s < n:
        z = jnp.zeros(a.shape[:-1] + (s,), a.dtype)
        a = a + jnp.concatenate([z, a[..., :n - s]], axis=-1)
        s *= 2
    return a


def _select_kernel(sbits_ref, mask_ref, idx_ref):
    bits = sbits_ref[...]

    def bs_body(_, carry):
        lo, hi = carry
        mid = lo + (hi - lo) // 2
        cnt = jnp.sum((bits >= mid).astype(jnp.int32), axis=1, keepdims=True)
        ge = cnt >= NK
        return jnp.where(ge, mid, lo), jnp.where(ge, hi, mid)

    lo0 = jnp.zeros((B, 1), jnp.int32)
    hi0 = jnp.full((B, 1), 0x3F800000, jnp.int32)
    thr, _ = lax.fori_loop(0, 31, bs_body, (lo0, hi0))

    m_gt = bits > thr
    m_eq = bits == thr
    cnt_gt = jnp.sum(m_gt.astype(jnp.int32), axis=1, keepdims=True)
    need = NK - cnt_gt
    eq_rank = _cumsum_lanes(m_eq.astype(jnp.int32))
    mask = m_gt | (m_eq & (eq_rank <= need))
    mask_ref[...] = mask

    c = _cumsum_lanes(mask.astype(jnp.int32))
    c_hi = c // 128
    c_lo = c - c_hi * 128
    iota16 = lax.broadcasted_iota(jnp.int32, (16, S), 0)
    iota128 = lax.broadcasted_iota(jnp.int32, (128, S), 0)
    for r in range(B):
        ehi = (c_hi[r:r + 1, :] == iota16).astype(jnp.float32)
        elo = (c_lo[r:r + 1, :] == iota128).astype(jnp.float32)
        h2d = lax.dot_general(ehi, elo, (((1,), (1,)), ((), ())),
                              preferred_element_type=jnp.float32)
        hist = h2d.reshape(1, 2048)
        counts = _cumsum_lanes(hist)
        idx_ref[r:r + 1, :] = counts[:, :NK_PAD].astype(jnp.int32)


def _gather_kernel(b_ref, i_ref, *refs):
    out_ref = refs[-1]
    rows = [refs[r][...].reshape(1, D) for r in range(W)]
    out_ref[...] = jnp.concatenate(rows, axis=0)


def _select(sbits):
    return pl.pallas_call(
        _select_kernel,
        out_shape=(
            jax.ShapeDtypeStruct((B, S), jnp.bool_),
            jax.ShapeDtypeStruct((B, NK_PAD), jnp.int32),
        ),
    )(sbits)


def _gather(b_arr, idx_flat, x):
    grid = (FLAT // W,)
    in_specs = [
        pl.BlockSpec((1, 1, 1, D),
                     (lambda g, b_ref, i_ref, r=r:
                      (b_ref[g * W + r], i_ref[g * W + r], 0, 0)))
        for r in range(W)
    ]
    out_specs = pl.BlockSpec((W, D), lambda g, b_ref, i_ref: (g, 0))
    return pl.pallas_call(
        _gather_kernel,
        grid_spec=pltpu.PrefetchScalarGridSpec(
            num_scalar_prefetch=2,
            grid=grid,
            in_specs=[in_specs[r] for r in range(W)],
            out_specs=out_specs,
        ),
        out_shape=jax.ShapeDtypeStruct((FLAT, D), jnp.float32),
    )(b_arr, idx_flat, *([x.reshape(B, S, 1, D)] * W))


def kernel(x):
    scores = jax.random.uniform(jax.random.key(42), (B, S))
    sbits = lax.bitcast_convert_type(scores, jnp.int32)
    mask, idx_pad = _select(sbits)
    idx = idx_pad[:, :NK]
    idx_flat = idx.reshape(FLAT)
    b_arr = jnp.repeat(jnp.arange(B, dtype=jnp.int32), NK)
    masked_flat = _gather(b_arr, idx_flat, x)
    masked = masked_flat.reshape(B, NK, D)
    return masked, mask


# speedup vs baseline: 6.2288x; 6.2288x over previous
import functools

import jax
import jax.numpy as jnp
from jax import lax
from jax.experimental import pallas as pl
from jax.experimental.pallas import tpu as pltpu
from jax.experimental.pallas import tpu_sc as plsc

B, S, D = 4, 8192, 1024
NK = 1638
NK_PAD = 2048
FLAT = B * NK
NW = 32
TPW = 208
CHS = (56, 56, 56, 40)
CHOFF = (0, 56, 112, 168)
NCHUNK = len(CHS)


def _start(w):
    return 8 * ((FLAT * w) // (8 * NW))


def _cumsum_lanes(a):
    n = a.shape[-1]
    s = 1
    while s < n:
        z = jnp.zeros(a.shape[:-1] + (s,), a.dtype)
        a = a + jnp.concatenate([z, a[..., :n - s]], axis=-1)
        s *= 2
    return a


def _select_kernel(sbits_ref, mask_ref, idx_ref):
    bits = sbits_ref[...]

    def bs_body(_, carry):
        lo, hi = carry
        mid = lo + (hi - lo) // 2
        cnt = jnp.sum((bits >= mid).astype(jnp.int32), axis=1, keepdims=True)
        ge = cnt >= NK
        return jnp.where(ge, mid, lo), jnp.where(ge, hi, mid)

    lo0 = jnp.zeros((B, 1), jnp.int32)
    hi0 = jnp.full((B, 1), 0x3F800000, jnp.int32)
    thr, _ = lax.fori_loop(0, 31, bs_body, (lo0, hi0))

    m_gt = bits > thr
    m_eq = bits == thr
    cnt_gt = jnp.sum(m_gt.astype(jnp.int32), axis=1, keepdims=True)
    need = NK - cnt_gt
    eq_rank = _cumsum_lanes(m_eq.astype(jnp.int32))
    mask = m_gt | (m_eq & (eq_rank <= need))
    mask_ref[...] = mask

    c = _cumsum_lanes(mask.astype(jnp.int32))
    c_hi = c // 128
    c_lo = c - c_hi * 128
    iota16 = lax.broadcasted_iota(jnp.int32, (16, S), 0)
    iota128 = lax.broadcasted_iota(jnp.int32, (128, S), 0)
    rows = []
    for r in range(B):
        ehi = (c_hi[r:r + 1, :] == iota16).astype(jnp.float32)
        elo = (c_lo[r:r + 1, :] == iota128).astype(jnp.float32)
        h2d = lax.dot_general(ehi, elo, (((1,), (1,)), ((), ())),
                              preferred_element_type=jnp.float32)
        hist = h2d.reshape(1, NK_PAD)
        counts = _cumsum_lanes(hist)
        rows.append(counts[:, :NK].astype(jnp.int32) + r * S)
    flat = jnp.concatenate(rows + [jnp.zeros((1, NW * TPW - FLAT), jnp.int32)],
                           axis=-1)
    table = jnp.concatenate(
        [flat[:, _start(w):_start(w) + TPW] for w in range(NW)],
        axis=-1)
    idx_ref[...] = table


def _select(sbits):
    return pl.pallas_call(
        _select_kernel,
        out_shape=(
            jax.ShapeDtypeStruct((B, S), jnp.bool_),
            jax.ShapeDtypeStruct((1, NW * TPW), jnp.int32),
        ),
    )(sbits)


def _gather_sc(idx2d, x2d):
    mesh = plsc.VectorSubcoreMesh(core_axis_name="c", subcore_axis_name="s")

    @functools.partial(
        pl.kernel, mesh=mesh,
        out_type=jax.ShapeDtypeStruct((FLAT, D), jnp.float32),
        scratch_types=[
            pltpu.VMEM((TPW,), jnp.int32),
            pltpu.VMEM((CHS[0], D), jnp.float32),
            pltpu.VMEM((CHS[0], D), jnp.float32),
            pltpu.SemaphoreType.DMA,
            pltpu.SemaphoreType.DMA,
        ],
    )
    def k(x_hbm, idx_hbm, out_hbm, idx_v, buf0, buf1, g0, g1):
        wid = lax.axis_index("s") * 2 + lax.axis_index("c")
        start = 8 * ((FLAT * wid) // (8 * NW))
        pltpu.sync_copy(idx_hbm.at[pl.ds(wid * TPW, TPW)], idx_v)
        bufs = (buf0, buf1)
        sems = (g0, g1)
        copies = []
        for c in range(NCHUNK):
            copies.append(pltpu.async_copy(
                x_hbm.at[idx_v.at[pl.ds(CHOFF[c], CHS[c])]],
                bufs[c % 2].at[pl.ds(0, CHS[c])], sems[c % 2]))
            if c == 0:
                continue
            copies[c - 1].wait()
            base = start + CHOFF[c - 1]
            pltpu.sync_copy(bufs[(c - 1) % 2].at[pl.ds(0, CHS[c - 1])],
                            out_hbm.at[pl.ds(base, CHS[c - 1])])
        copies[NCHUNK - 1].wait()
        pltpu.sync_copy(bufs[(NCHUNK - 1) % 2].at[pl.ds(0, CHS[NCHUNK - 1])],
                        out_hbm.at[pl.ds(start + CHOFF[NCHUNK - 1],
                                         CHS[NCHUNK - 1])])

    return k(x2d, idx2d)


def kernel(x):
    scores = jax.random.uniform(jax.random.key(42), (B, S))
    sbits = lax.bitcast_convert_type(scores, jnp.int32)
    mask, idx2d = _select(sbits)
    masked_flat = _gather_sc(idx2d.reshape(NW * TPW), x.reshape(B * S, D))
    return masked_flat.reshape(B, NK, D), mask


# speedup vs baseline: 6.3989x; 1.0273x over previous
import functools

import jax
import jax.numpy as jnp
from jax import lax
from jax.experimental import pallas as pl
from jax.experimental.pallas import tpu as pltpu
from jax.experimental.pallas import tpu_sc as plsc

B, S, D = 4, 8192, 1024
NK = 1638
NK_PAD = 2048
FLAT = B * NK
NW = 32
TPW = 208
CHS = (56, 56, 56, 40)
CHOFF = (0, 56, 112, 168)
NCHUNK = len(CHS)
SKJ = tuple(8 * ((NK * j) // 64) for j in range(8))
REM7 = NK - SKJ[7] - CHOFF[NCHUNK - 1]


def _cumsum_lanes(a):
    n = a.shape[-1]
    s = 1
    while s < n:
        z = jnp.zeros(a.shape[:-1] + (s,), a.dtype)
        a = a + jnp.concatenate([z, a[..., :n - s]], axis=-1)
        s *= 2
    return a


def _select_kernel(sbits_ref, mask_ref, idx_ref):
    bits = sbits_ref[...]

    def bs_body(_, carry):
        lo, hi = carry
        mid = lo + (hi - lo) // 2
        cnt = jnp.sum((bits >= mid).astype(jnp.int32), axis=1, keepdims=True)
        ge = cnt >= NK
        return jnp.where(ge, mid, lo), jnp.where(ge, hi, mid)

    lo0 = jnp.zeros((B, 1), jnp.int32)
    hi0 = jnp.full((B, 1), 0x3F800000, jnp.int32)
    thr, _ = lax.fori_loop(0, 31, bs_body, (lo0, hi0))

    m_gt = bits > thr
    m_eq = bits == thr
    cnt_gt = jnp.sum(m_gt.astype(jnp.int32), axis=1, keepdims=True)
    need = NK - cnt_gt
    eq_rank = _cumsum_lanes(m_eq.astype(jnp.int32))
    mask = m_gt | (m_eq & (eq_rank <= need))
    mask_ref[...] = mask

    c = _cumsum_lanes(mask.astype(jnp.int32))
    c_hi = c // 128
    c_lo = c - c_hi * 128
    iota16 = lax.broadcasted_iota(jnp.int32, (16, S), 0)
    iota128 = lax.broadcasted_iota(jnp.int32, (128, S), 0)
    rows = []
    for r in range(B):
        ehi = (c_hi[r:r + 1, :] == iota16).astype(jnp.float32)
        elo = (c_lo[r:r + 1, :] == iota128).astype(jnp.float32)
        h2d = lax.dot_general(ehi, elo, (((1,), (1,)), ((), ())),
                              preferred_element_type=jnp.float32)
        hist = h2d.reshape(1, NK_PAD)
        counts = _cumsum_lanes(hist)
        rows.append(counts[:, :NK].astype(jnp.int32) + r * S)
    parts = []
    for r in range(B):
        ext = jnp.concatenate(
            [rows[r], rows[r][:, NK - 1:NK], rows[r][:, NK - 1:NK]], axis=-1)
        parts.extend(ext[:, SKJ[j]:SKJ[j] + TPW] for j in range(8))
    idx_ref[...] = jnp.concatenate(parts, axis=-1)


def _select(sbits):
    return pl.pallas_call(
        _select_kernel,
        out_shape=(
            jax.ShapeDtypeStruct((B, S), jnp.bool_),
            jax.ShapeDtypeStruct((1, NW * TPW), jnp.int32),
        ),
    )(sbits)


def _gather_sc(idx2d, x2d):
    mesh = plsc.VectorSubcoreMesh(core_axis_name="c", subcore_axis_name="s")

    @functools.partial(
        pl.kernel, mesh=mesh,
        out_type=jax.ShapeDtypeStruct((B, NK, D), jnp.float32),
        scratch_types=[
            pltpu.VMEM((TPW,), jnp.int32),
            pltpu.VMEM((CHS[0], D), jnp.float32),
            pltpu.VMEM((CHS[0], D), jnp.float32),
            pltpu.SemaphoreType.DMA,
            pltpu.SemaphoreType.DMA,
        ],
    )
    def k(x_hbm, idx_hbm, out_hbm, idx_v, buf0, buf1, g0, g1):
        wid = lax.axis_index("s") * 2 + lax.axis_index("c")
        b = wid // 8
        j = wid - b * 8
        start = 8 * ((NK * j) // 64)
        pltpu.sync_copy(idx_hbm.at[pl.ds(wid * TPW, TPW)], idx_v)
        bufs = (buf0, buf1)
        sems = (g0, g1)
        copies = []
        for c in range(NCHUNK):
            copies.append(pltpu.async_copy(
                x_hbm.at[idx_v.at[pl.ds(CHOFF[c], CHS[c])]],
                bufs[c % 2].at[pl.ds(0, CHS[c])], sems[c % 2]))
            if c == 0:
                continue
            copies[c - 1].wait()
            pltpu.sync_copy(bufs[(c - 1) % 2].at[pl.ds(0, CHS[c - 1])],
                            out_hbm.at[b, pl.ds(start + CHOFF[c - 1],
                                                CHS[c - 1])])
        copies[NCHUNK - 1].wait()
        pltpu.sync_copy(bufs[(NCHUNK - 1) % 2].at[pl.ds(0, CHS[NCHUNK - 1])],
                        out_hbm.at[b, pl.ds(start + CHOFF[NCHUNK - 1],
                                            CHS[NCHUNK - 1])])

    return k(x2d, idx2d)


def kernel(x):
    scores = jax.random.uniform(jax.random.key(42), (B, S))
    sbits = lax.bitcast_convert_type(scores, jnp.int32)
    mask, idx2d = _select(sbits)
    masked = _gather_sc(idx2d.reshape(NW * TPW), x.reshape(B * S, D))
    return masked, mask


# speedup vs baseline: 9.1770x; 1.4341x over previous
import functools

import jax
import jax.numpy as jnp
import numpy as np
from jax import lax
from jax.experimental import pallas as pl
from jax.experimental.pallas import tpu as pltpu
from jax.experimental.pallas import tpu_sc as plsc

B, S, D = 4, 8192, 1024
NK = 1638
NK_PAD = 2048
FLAT = B * NK
NW = 32
KCH = (14, 14, 14, 10)
KOFF = (0, 14, 28, 42)
NCHUNK = len(KCH)
KPW = 52
TPW = B * KPW

_KW = [(NK * w) // NW for w in range(NW)]
_P = np.empty((NW * TPW,), np.int32)
for _w in range(NW):
    for _i in range(KPW):
        for _b in range(B):
            _P[_w * TPW + B * _i + _b] = _b * NK_PAD + _KW[_w] + _i


def _cumsum_lanes(a):
    n = a.shape[-1]
    s = 1
    while s < n:
        z = jnp.zeros(a.shape[:-1] + (s,), a.dtype)
        a = a + jnp.concatenate([z, a[..., :n - s]], axis=-1)
        s *= 2
    return a


def _select_kernel(sbits_ref, mask_ref, idx_ref):
    bits = sbits_ref[...]

    def bs_body(_, carry):
        lo, hi = carry
        mid = lo + (hi - lo) // 2
        cnt = jnp.sum((bits >= mid).astype(jnp.int32), axis=1, keepdims=True)
        ge = cnt >= NK
        return jnp.where(ge, mid, lo), jnp.where(ge, hi, mid)

    lo0 = jnp.zeros((B, 1), jnp.int32)
    hi0 = jnp.full((B, 1), 0x3F800000, jnp.int32)
    thr, _ = lax.fori_loop(0, 31, bs_body, (lo0, hi0))

    m_gt = bits > thr
    m_eq = bits == thr
    cnt_gt = jnp.sum(m_gt.astype(jnp.int32), axis=1, keepdims=True)
    need = NK - cnt_gt
    eq_rank = _cumsum_lanes(m_eq.astype(jnp.int32))
    mask = m_gt | (m_eq & (eq_rank <= need))
    mask_ref[...] = mask

    c = _cumsum_lanes(mask.astype(jnp.int32))
    c_hi = c // 128
    c_lo = c - c_hi * 128
    iota16 = lax.broadcasted_iota(jnp.int32, (16, S), 0)
    iota128 = lax.broadcasted_iota(jnp.int32, (128, S), 0)
    for r in range(B):
        ehi = (c_hi[r:r + 1, :] == iota16).astype(jnp.float32)
        elo = (c_lo[r:r + 1, :] == iota128).astype(jnp.float32)
        h2d = lax.dot_general(ehi, elo, (((1,), (1,)), ((), ())),
                              preferred_element_type=jnp.float32)
        hist = h2d.reshape(1, NK_PAD)
        counts = _cumsum_lanes(hist)
        idx_ref[r:r + 1, :] = counts.astype(jnp.int32) + r * S


def _select(sbits):
    return pl.pallas_call(
        _select_kernel,
        out_shape=(
            jax.ShapeDtypeStruct((B, S), jnp.bool_),
            jax.ShapeDtypeStruct((B, NK_PAD), jnp.int32),
        ),
    )(sbits)


def _gather_sc(table, x2d):
    mesh = plsc.VectorSubcoreMesh(core_axis_name="c", subcore_axis_name="s")

    @functools.partial(
        pl.kernel, mesh=mesh,
        out_type=jax.ShapeDtypeStruct((NK, B, D), jnp.float32),
        scratch_types=[
            pltpu.VMEM((TPW,), jnp.int32),
            pltpu.VMEM((B * KCH[0], D), jnp.float32),
            pltpu.VMEM((B * KCH[0], D), jnp.float32),
            pltpu.SemaphoreType.DMA,
            pltpu.SemaphoreType.DMA,
            pltpu.SemaphoreType.DMA,
            pltpu.SemaphoreType.DMA,
        ],
    )
    def k(x_hbm, idx_hbm, out_hbm, idx_v, buf0, buf1, g0, g1, s0, s1):
        wid = lax.axis_index("s") * 2 + lax.axis_index("c")
        kstart = (NK * wid) // NW
        pltpu.sync_copy(idx_hbm.at[pl.ds(wid * TPW, TPW)], idx_v)
        bufs = (buf0, buf1)
        gsems = (g0, g1)
        ssems = (s0, s1)

        def issue_gather(c):
            return pltpu.async_copy(
                x_hbm.at[idx_v.at[pl.ds(B * KOFF[c], B * KCH[c])]],
                bufs[c % 2].at[pl.ds(0, B * KCH[c])], gsems[c % 2])

        def issue_stores(c):
            return [pltpu.async_copy(
                bufs[c % 2].at[pl.ds(B * i, B)],
                out_hbm.at[kstart + KOFF[c] + i], ssems[c % 2])
                for i in range(KCH[c])]

        gathers = [issue_gather(0)]
        stores = []
        for c in range(NCHUNK):
            gathers[c].wait()
            stores.append(issue_stores(c))
            if c + 1 < NCHUNK:
                if c >= 1:
                    for h in stores[c - 1]:
                        h.wait()
                gathers.append(issue_gather(c + 1))
        for cs in stores[NCHUNK - 2:]:
            for h in cs:
                h.wait()

    return k(x2d, table)


def kernel(x):
    scores = jax.random.uniform(jax.random.key(42), (B, S))
    sbits = lax.bitcast_convert_type(scores, jnp.int32)
    mask, idx_all = _select(sbits)
    table = jnp.take(idx_all.reshape(B * NK_PAD), _P)
    out_kbd = _gather_sc(table, x.reshape(B * S, D))
    return jnp.swapaxes(out_kbd, 0, 1), mask


# speedup vs baseline: 9.7086x; 1.0579x over previous
import functools

import jax
import jax.numpy as jnp
import numpy as np
from jax import lax
from jax.experimental import pallas as pl
from jax.experimental.pallas import tpu as pltpu
from jax.experimental.pallas import tpu_sc as plsc

B, S, D = 4, 8192, 1024
NK = 1638
NK_PAD = 2048
FLAT = B * NK
NW = 32
KCH = (8, 8, 8, 8, 8, 8, 8)
KOFF = (0, 8, 16, 24, 32, 40, 48)
NCHUNK = len(KCH)
KPW = 56


def _cumsum_lanes(a):
    n = a.shape[-1]
    s = 1
    while s < n:
        z = jnp.zeros(a.shape[:-1] + (s,), a.dtype)
        a = a + jnp.concatenate([z, a[..., :n - s]], axis=-1)
        s *= 2
    return a


def _select_kernel(sbits_ref, mask_ref, idx_ref):
    bits = sbits_ref[...]

    def bs_body(_, carry):
        lo, hi = carry
        mid = lo + (hi - lo) // 2
        cnt = jnp.sum((bits >= mid).astype(jnp.int32), axis=1, keepdims=True)
        ge = cnt >= NK
        return jnp.where(ge, mid, lo), jnp.where(ge, hi, mid)

    lo0 = jnp.zeros((B, 1), jnp.int32)
    hi0 = jnp.full((B, 1), 0x3F800000, jnp.int32)
    thr, _ = lax.fori_loop(0, 31, bs_body, (lo0, hi0))

    m_gt = bits > thr
    m_eq = bits == thr
    cnt_gt = jnp.sum(m_gt.astype(jnp.int32), axis=1, keepdims=True)
    need = NK - cnt_gt
    eq_rank = _cumsum_lanes(m_eq.astype(jnp.int32))
    mask = m_gt | (m_eq & (eq_rank <= need))
    mask_ref[...] = mask

    c = _cumsum_lanes(mask.astype(jnp.int32))
    c_hi = c // 128
    c_lo = c - c_hi * 128
    iota16 = lax.broadcasted_iota(jnp.int32, (16, S), 0)
    iota128 = lax.broadcasted_iota(jnp.int32, (128, S), 0)
    allrows = []
    for r in range(B):
        ehi = (c_hi[r:r + 1, :] == iota16).astype(jnp.float32)
        elo = (c_lo[r:r + 1, :] == iota128).astype(jnp.float32)
        h2d = lax.dot_general(ehi, elo, (((1,), (1,)), ((), ())),
                              preferred_element_type=jnp.float32)
        hist = h2d.reshape(1, NK_PAD)
        counts = _cumsum_lanes(hist)
        allrows.append(jnp.minimum(counts.astype(jnp.int32), S - 1) + r * S)
    idx_ref[...] = jnp.concatenate(allrows, axis=0)


def _select(sbits):
    return pl.pallas_call(
        _select_kernel,
        out_shape=(
            jax.ShapeDtypeStruct((B, S), jnp.bool_),
            jax.ShapeDtypeStruct((B, NK_PAD), jnp.int32),
        ),
    )(sbits)


def _gather_sc(idx_t, x2d):
    mesh = plsc.VectorSubcoreMesh(core_axis_name="c", subcore_axis_name="s")

    @functools.partial(
        pl.kernel, mesh=mesh,
        out_type=jax.ShapeDtypeStruct((NK, B, D), jnp.float32),
        scratch_types=[
            pltpu.VMEM((B * NK_PAD,), jnp.int32),
            pltpu.VMEM((B * KCH[0], D), jnp.float32),
            pltpu.VMEM((B * KCH[0], D), jnp.float32),
            pltpu.SemaphoreType.DMA,
            pltpu.SemaphoreType.DMA,
            pltpu.SemaphoreType.DMA,
            pltpu.SemaphoreType.DMA,
        ],
    )
    def k(x_hbm, idx_hbm, out_hbm, ids_v, buf0, buf1, g0, g1, s0, s1):
        wid = lax.axis_index("s") * 2 + lax.axis_index("c")
        kstart = 8 * ((NK * wid) // (8 * NW))
        pltpu.sync_copy(idx_hbm, ids_v)
        bufs = (buf0, buf1)
        gsems = (g0, g1)
        ssems = (s0, s1)

        def issue_gather(c):
            return pltpu.async_copy(
                x_hbm.at[ids_v.at[pl.ds(B * (kstart + KOFF[c]), B * KCH[c])]],
                bufs[c % 2].at[pl.ds(0, B * KCH[c])], gsems[c % 2])

        def issue_stores(c):
            for i in range(KCH[c]):
                kpos = kstart + KOFF[c] + i

                @pl.when(kpos < NK)
                def _(i=i, kpos=kpos, c=c):
                    pltpu.async_copy(bufs[c % 2].at[pl.ds(B * i, B)],
                                     out_hbm.at[kpos], ssems[c % 2])

        def drain_stores(c):
            for i in range(KCH[c]):
                kpos = kstart + KOFF[c] + i

                @pl.when(kpos < NK)
                def _(i=i, kpos=kpos, c=c):
                    pltpu.make_async_copy(bufs[c % 2].at[pl.ds(B * i, B)],
                                          out_hbm.at[kpos],
                                          ssems[c % 2]).wait()

        gathers = [issue_gather(0)]
        for c in range(NCHUNK):
            gathers[c].wait()
            issue_stores(c)
            if c + 1 < NCHUNK:
                if c >= 1:
                    drain_stores(c - 1)
                gathers.append(issue_gather(c + 1))
        drain_stores(NCHUNK - 2)
        drain_stores(NCHUNK - 1)

    return k(x2d, jnp.transpose(idx_t).reshape(B * NK_PAD))


def kernel(x):
    scores = jax.random.uniform(jax.random.key(42), (B, S))
    sbits = lax.bitcast_convert_type(scores, jnp.int32)
    mask, idx_t = _select(sbits)
    out_kbd = _gather_sc(idx_t, x.reshape(B * S, D))
    return jnp.swapaxes(out_kbd, 0, 1), mask


# speedup vs baseline: 9.9725x; 1.0272x over previous
import functools

import jax
import jax.numpy as jnp
import numpy as np
from jax import lax
from jax.experimental import pallas as pl
from jax.experimental.pallas import tpu as pltpu
from jax.experimental.pallas import tpu_sc as plsc

B, S, D = 4, 8192, 1024
NK = 1638
NK_PAD = 2048
FLAT = B * NK
NW = 32
KCH = (14, 14, 14, 14)
KOFF = (0, 14, 28, 42)
NCHUNK = len(KCH)
KPW = 56


def _cumsum_lanes(a):
    n = a.shape[-1]
    s = 1
    while s < n:
        z = jnp.zeros(a.shape[:-1] + (s,), a.dtype)
        a = a + jnp.concatenate([z, a[..., :n - s]], axis=-1)
        s *= 2
    return a


def _select_kernel(sbits_ref, mask_ref, idx_ref):
    bits = sbits_ref[...]

    def bs_body(_, carry):
        lo, hi = carry
        mid = lo + (hi - lo) // 2
        cnt = jnp.sum((bits >= mid).astype(jnp.int32), axis=1, keepdims=True)
        ge = cnt >= NK
        return jnp.where(ge, mid, lo), jnp.where(ge, hi, mid)

    lo0 = jnp.zeros((B, 1), jnp.int32)
    hi0 = jnp.full((B, 1), 0x3F800000, jnp.int32)
    thr, _ = lax.fori_loop(0, 31, bs_body, (lo0, hi0))

    m_gt = bits > thr
    m_eq = bits == thr
    cnt_gt = jnp.sum(m_gt.astype(jnp.int32), axis=1, keepdims=True)
    need = NK - cnt_gt
    eq_rank = _cumsum_lanes(m_eq.astype(jnp.int32))
    mask = m_gt | (m_eq & (eq_rank <= need))
    mask_ref[...] = mask

    c = _cumsum_lanes(mask.astype(jnp.int32))
    c_hi = c // 128
    c_lo = c - c_hi * 128
    iota16 = lax.broadcasted_iota(jnp.int32, (16, S), 0)
    iota128 = lax.broadcasted_iota(jnp.int32, (128, S), 0)
    allrows = []
    for r in range(B):
        ehi = (c_hi[r:r + 1, :] == iota16).astype(jnp.float32)
        elo = (c_lo[r:r + 1, :] == iota128).astype(jnp.float32)
        h2d = lax.dot_general(ehi, elo, (((1,), (1,)), ((), ())),
                              preferred_element_type=jnp.float32)
        hist = h2d.reshape(1, NK_PAD)
        counts = _cumsum_lanes(hist)
        allrows.append(jnp.minimum(counts.astype(jnp.int32), S - 1) + r * S)
    idx_ref[...] = jnp.concatenate(allrows, axis=0)


def _select(sbits):
    return pl.pallas_call(
        _select_kernel,
        out_shape=(
            jax.ShapeDtypeStruct((B, S), jnp.bool_),
            jax.ShapeDtypeStruct((B, NK_PAD), jnp.int32),
        ),
    )(sbits)


def _gather_sc(idx_t, x2d):
    mesh = plsc.VectorSubcoreMesh(core_axis_name="c", subcore_axis_name="s")

    @functools.partial(
        pl.kernel, mesh=mesh,
        out_type=jax.ShapeDtypeStruct((NK, B, D), jnp.float32),
        scratch_types=[
            pltpu.VMEM((B * NK_PAD,), jnp.int32),
            pltpu.VMEM((B * KCH[0], D), jnp.float32),
            pltpu.VMEM((B * KCH[0], D), jnp.float32),
            pltpu.SemaphoreType.DMA,
            pltpu.SemaphoreType.DMA,
            pltpu.SemaphoreType.DMA,
            pltpu.SemaphoreType.DMA,
        ],
    )
    def k(x_hbm, idx_hbm, out_hbm, ids_v, buf0, buf1, g0, g1, s0, s1):
        wid = lax.axis_index("s") * 2 + lax.axis_index("c")
        kstart = 8 * ((NK * wid) // (8 * NW))
        pltpu.sync_copy(idx_hbm, ids_v)
        bufs = (buf0, buf1)
        gsems = (g0, g1)
        ssems = (s0, s1)

        def issue_gather(c):
            return pltpu.async_copy(
                x_hbm.at[ids_v.at[pl.ds(B * (kstart + KOFF[c]), B * KCH[c])]],
                bufs[c % 2].at[pl.ds(0, B * KCH[c])], gsems[c % 2])

        def issue_stores(c):
            for i in range(KCH[c]):
                kpos = kstart + KOFF[c] + i
                if c < NCHUNK - 1:
                    pltpu.async_copy(bufs[c % 2].at[pl.ds(B * i, B)],
                                     out_hbm.at[kpos], ssems[c % 2])
                else:
                    @pl.when(kpos < NK)
                    def _(i=i, kpos=kpos, c=c):
                        pltpu.async_copy(bufs[c % 2].at[pl.ds(B * i, B)],
                                         out_hbm.at[kpos], ssems[c % 2])

        def drain_stores(c):
            for i in range(KCH[c]):
                kpos = kstart + KOFF[c] + i
                if c < NCHUNK - 1:
                    pltpu.make_async_copy(bufs[c % 2].at[pl.ds(B * i, B)],
                                          out_hbm.at[kpos],
                                          ssems[c % 2]).wait()
                else:
                    @pl.when(kpos < NK)
                    def _(i=i, kpos=kpos, c=c):
                        pltpu.make_async_copy(bufs[c % 2].at[pl.ds(B * i, B)],
                                              out_hbm.at[kpos],
                                              ssems[c % 2]).wait()

        gathers = [issue_gather(0)]
        for c in range(NCHUNK):
            gathers[c].wait()
            issue_stores(c)
            if c + 1 < NCHUNK:
                if c >= 1:
                    drain_stores(c - 1)
                gathers.append(issue_gather(c + 1))
        drain_stores(NCHUNK - 2)
        drain_stores(NCHUNK - 1)

    return k(x2d, jnp.transpose(idx_t).reshape(B * NK_PAD))


def kernel(x):
    scores = jax.random.uniform(jax.random.key(42), (B, S))
    sbits = lax.bitcast_convert_type(scores, jnp.int32)
    mask, idx_t = _select(sbits)
    out_kbd = _gather_sc(idx_t, x.reshape(B * S, D))
    return jnp.swapaxes(out_kbd, 0, 1), mask


# speedup vs baseline: 10.3345x; 1.0363x over previous
import functools

import jax
import jax.numpy as jnp
import numpy as np
from jax import lax
from jax.experimental import pallas as pl
from jax.experimental.pallas import tpu as pltpu
from jax.experimental.pallas import tpu_sc as plsc

B, S, D = 4, 8192, 1024
NK = 1638
NK_PAD = 2048
FLAT = B * NK
NW = 32
KCH = (14, 14, 14, 14)
KOFF = (0, 14, 28, 42)
NCHUNK = len(KCH)
KPW = 56


def _cumsum_lanes(a):
    n = a.shape[-1]
    s = 1
    while s < n:
        z = jnp.zeros(a.shape[:-1] + (s,), a.dtype)
        a = a + jnp.concatenate([z, a[..., :n - s]], axis=-1)
        s *= 2
    return a


def _select_kernel(sbits_ref, mask_ref, idx_ref):
    bits = sbits_ref[...]

    def bs_body(_, carry):
        lo, hi = carry
        mid = lo + (hi - lo) // 2
        cnt = jnp.sum((bits >= mid).astype(jnp.int32), axis=1, keepdims=True)
        ge = cnt >= NK
        return jnp.where(ge, mid, lo), jnp.where(ge, hi, mid)

    lo0 = jnp.full((B, 1), 0x3F400000, jnp.int32)
    hi0 = jnp.full((B, 1), 0x3F600000, jnp.int32)
    thr, _ = lax.fori_loop(0, 21, bs_body, (lo0, hi0))

    mask = bits >= thr
    mask_ref[...] = mask

    c = _cumsum_lanes(mask.astype(jnp.int32))
    c_hi = c // 128
    c_lo = c - c_hi * 128
    iota16 = lax.broadcasted_iota(jnp.int32, (16, S), 0)
    iota128 = lax.broadcasted_iota(jnp.int32, (128, S), 0)
    hists = []
    for r in range(B):
        ehi = (c_hi[r:r + 1, :] == iota16).astype(jnp.float32)
        elo = (c_lo[r:r + 1, :] == iota128).astype(jnp.float32)
        h2d = lax.dot_general(ehi, elo, (((1,), (1,)), ((), ())),
                              preferred_element_type=jnp.float32)
        hists.append(h2d.reshape(1, NK_PAD))
    counts = _cumsum_lanes(jnp.concatenate(hists, axis=0))
    ids = (jnp.minimum(counts.astype(jnp.int32), S - 1)
           + S * lax.broadcasted_iota(jnp.int32, (B, NK_PAD), 0))
    idx_ref[...] = ids


def _select(sbits):
    return pl.pallas_call(
        _select_kernel,
        out_shape=(
            jax.ShapeDtypeStruct((B, S), jnp.bool_),
            jax.ShapeDtypeStruct((B, NK_PAD), jnp.int32),
        ),
    )(sbits)


def _gather_sc(idx_t, x2d):
    mesh = plsc.VectorSubcoreMesh(core_axis_name="c", subcore_axis_name="s")

    @functools.partial(
        pl.kernel, mesh=mesh,
        out_type=jax.ShapeDtypeStruct((NK, B, D), jnp.float32),
        scratch_types=[
            pltpu.VMEM((B * NK_PAD,), jnp.int32),
            pltpu.VMEM((B * KCH[0], D), jnp.float32),
            pltpu.VMEM((B * KCH[0], D), jnp.float32),
            pltpu.SemaphoreType.DMA,
            pltpu.SemaphoreType.DMA,
            pltpu.SemaphoreType.DMA,
            pltpu.SemaphoreType.DMA,
        ],
    )
    def k(x_hbm, idx_hbm, out_hbm, ids_v, buf0, buf1, g0, g1, s0, s1):
        wid = lax.axis_index("s") * 2 + lax.axis_index("c")
        kstart = 8 * ((NK * wid) // (8 * NW))
        pltpu.sync_copy(idx_hbm, ids_v)
        bufs = (buf0, buf1)
        gsems = (g0, g1)
        ssems = (s0, s1)

        def issue_gather(c):
            return pltpu.async_copy(
                x_hbm.at[ids_v.at[pl.ds(B * (kstart + KOFF[c]), B * KCH[c])]],
                bufs[c % 2].at[pl.ds(0, B * KCH[c])], gsems[c % 2])

        def issue_stores(c):
            for i in range(KCH[c]):
                kpos = kstart + KOFF[c] + i
                if c < NCHUNK - 1:
                    pltpu.async_copy(bufs[c % 2].at[pl.ds(B * i, B)],
                                     out_hbm.at[kpos], ssems[c % 2])
                else:
                    @pl.when(kpos < NK)
                    def _(i=i, kpos=kpos, c=c):
                        pltpu.async_copy(bufs[c % 2].at[pl.ds(B * i, B)],
                                         out_hbm.at[kpos], ssems[c % 2])

        def drain_stores(c):
            for i in range(KCH[c]):
                kpos = kstart + KOFF[c] + i
                if c < NCHUNK - 1:
                    pltpu.make_async_copy(bufs[c % 2].at[pl.ds(B * i, B)],
                                          out_hbm.at[kpos],
                                          ssems[c % 2]).wait()
                else:
                    @pl.when(kpos < NK)
                    def _(i=i, kpos=kpos, c=c):
                        pltpu.make_async_copy(bufs[c % 2].at[pl.ds(B * i, B)],
                                              out_hbm.at[kpos],
                                              ssems[c % 2]).wait()

        gathers = [issue_gather(0)]
        for c in range(NCHUNK):
            gathers[c].wait()
            issue_stores(c)
            if c + 1 < NCHUNK:
                if c >= 1:
                    drain_stores(c - 1)
                gathers.append(issue_gather(c + 1))
        drain_stores(NCHUNK - 2)
        drain_stores(NCHUNK - 1)

    return k(x2d, jnp.transpose(idx_t).reshape(B * NK_PAD))


def kernel(x):
    scores = jax.random.uniform(jax.random.key(42), (B, S))
    sbits = lax.bitcast_convert_type(scores, jnp.int32)
    mask, idx_t = _select(sbits)
    out_kbd = _gather_sc(idx_t, x.reshape(B * S, D))
    return jnp.swapaxes(out_kbd, 0, 1), mask


# speedup vs baseline: 10.7172x; 1.0370x over previous
import functools

import jax
import jax.numpy as jnp
import numpy as np
from jax import lax
from jax.experimental import pallas as pl
from jax.experimental.pallas import tpu as pltpu
from jax.experimental.pallas import tpu_sc as plsc

B, S, D = 4, 8192, 1024
NK = 1638
NK_PAD = 2048
FLAT = B * NK
NW = 32
KCH = (14, 14, 14, 14)
KOFF = (0, 14, 28, 42)
NCHUNK = len(KCH)
KPW = 56


def _cumsum_lanes(a):
    n = a.shape[-1]
    s = 1
    while s < n:
        z = jnp.zeros(a.shape[:-1] + (s,), a.dtype)
        a = a + jnp.concatenate([z, a[..., :n - s]], axis=-1)
        s *= 2
    return a


def _select_kernel(mask_ref, idx_ref):
    i = (S * lax.broadcasted_iota(jnp.uint32, (B, S), 0)
         + lax.broadcasted_iota(jnp.uint32, (B, S), 1))
    rots = ((13, 15, 26, 6), (17, 29, 16, 24))
    ks = (jnp.uint32(0), jnp.uint32(42), jnp.uint32(0 ^ 42 ^ 0x1BD11BDA))
    x0 = jnp.zeros((B, S), jnp.uint32) + ks[0]
    x1 = i + ks[1]

    def rounds(x0, x1, rs):
        for r in rs:
            x0 = x0 + x1
            x1 = x0 ^ ((x1 << jnp.uint32(r)) | (x1 >> jnp.uint32(32 - r)))
        return x0, x1

    x0, x1 = rounds(x0, x1, rots[0])
    x0, x1 = x0 + ks[1], x1 + ks[2] + jnp.uint32(1)
    x0, x1 = rounds(x0, x1, rots[1])
    x0, x1 = x0 + ks[2], x1 + ks[0] + jnp.uint32(2)
    x0, x1 = rounds(x0, x1, rots[0])
    x0, x1 = x0 + ks[0], x1 + ks[1] + jnp.uint32(3)
    x0, x1 = rounds(x0, x1, rots[1])
    x0, x1 = x0 + ks[1], x1 + ks[2] + jnp.uint32(4)
    x0, x1 = rounds(x0, x1, rots[0])
    x0, x1 = x0 + ks[2], x1 + ks[0] + jnp.uint32(5)
    bits = ((x0 ^ x1) >> jnp.uint32(9)).astype(jnp.int32)

    def bs_body(_, carry):
        lo, hi = carry
        mid = lo + (hi - lo) // 2
        cnt = jnp.sum((bits >= mid).astype(jnp.int32), axis=1, keepdims=True)
        ge = cnt >= NK
        return jnp.where(ge, mid, lo), jnp.where(ge, hi, mid)

    lo0 = jnp.full((B, 1), 3 << 21, jnp.int32)
    hi0 = jnp.full((B, 1), 7 << 20, jnp.int32)
    thr, _ = lax.fori_loop(0, 20, bs_body, (lo0, hi0))

    mask = bits >= thr
    mask_ref[...] = mask

    c = _cumsum_lanes(mask.astype(jnp.int32))
    c_hi = c // 128
    c_lo = c - c_hi * 128
    iota16 = lax.broadcasted_iota(jnp.int32, (16, S), 0)
    iota128 = lax.broadcasted_iota(jnp.int32, (128, S), 0)
    hists = []
    for r in range(B):
        ehi = (c_hi[r:r + 1, :] == iota16).astype(jnp.float32)
        elo = (c_lo[r:r + 1, :] == iota128).astype(jnp.float32)
        h2d = lax.dot_general(ehi, elo, (((1,), (1,)), ((), ())),
                              preferred_element_type=jnp.float32)
        hists.append(h2d.reshape(1, NK_PAD))
    counts = _cumsum_lanes(jnp.concatenate(hists, axis=0))
    ids = (jnp.minimum(counts.astype(jnp.int32), S - 1)
           + S * lax.broadcasted_iota(jnp.int32, (B, NK_PAD), 0))
    idx_ref[...] = ids


def _select():
    return pl.pallas_call(
        _select_kernel,
        out_shape=(
            jax.ShapeDtypeStruct((B, S), jnp.bool_),
            jax.ShapeDtypeStruct((B, NK_PAD), jnp.int32),
        ),
    )()


def _gather_sc(idx_t, x2d):
    mesh = plsc.VectorSubcoreMesh(core_axis_name="c", subcore_axis_name="s")

    @functools.partial(
        pl.kernel, mesh=mesh,
        out_type=jax.ShapeDtypeStruct((NK, B, D), jnp.float32),
        scratch_types=[
            pltpu.VMEM((B * NK_PAD,), jnp.int32),
            pltpu.VMEM((B * KCH[0], D), jnp.float32),
            pltpu.VMEM((B * KCH[0], D), jnp.float32),
            pltpu.SemaphoreType.DMA,
            pltpu.SemaphoreType.DMA,
            pltpu.SemaphoreType.DMA,
            pltpu.SemaphoreType.DMA,
        ],
    )
    def k(x_hbm, idx_hbm, out_hbm, ids_v, buf0, buf1, g0, g1, s0, s1):
        wid = lax.axis_index("s") * 2 + lax.axis_index("c")
        kstart = 8 * ((NK * wid) // (8 * NW))
        pltpu.sync_copy(idx_hbm, ids_v)
        bufs = (buf0, buf1)
        gsems = (g0, g1)
        ssems = (s0, s1)

        def issue_gather(c):
            return pltpu.async_copy(
                x_hbm.at[ids_v.at[pl.ds(B * (kstart + KOFF[c]), B * KCH[c])]],
                bufs[c % 2].at[pl.ds(0, B * KCH[c])], gsems[c % 2])

        def issue_stores(c):
            for i in range(KCH[c]):
                kpos = kstart + KOFF[c] + i
                if c < NCHUNK - 1:
                    pltpu.async_copy(bufs[c % 2].at[pl.ds(B * i, B)],
                                     out_hbm.at[kpos], ssems[c % 2])
                else:
                    @pl.when(kpos < NK)
                    def _(i=i, kpos=kpos, c=c):
                        pltpu.async_copy(bufs[c % 2].at[pl.ds(B * i, B)],
                                         out_hbm.at[kpos], ssems[c % 2])

        def drain_stores(c):
            for i in range(KCH[c]):
                kpos = kstart + KOFF[c] + i
                if c < NCHUNK - 1:
                    pltpu.make_async_copy(bufs[c % 2].at[pl.ds(B * i, B)],
                                          out_hbm.at[kpos],
                                          ssems[c % 2]).wait()
                else:
                    @pl.when(kpos < NK)
                    def _(i=i, kpos=kpos, c=c):
                        pltpu.make_async_copy(bufs[c % 2].at[pl.ds(B * i, B)],
                                              out_hbm.at[kpos],
                                              ssems[c % 2]).wait()

        gathers = [issue_gather(0)]
        for c in range(NCHUNK):
            gathers[c].wait()
            issue_stores(c)
            if c + 1 < NCHUNK:
                if c >= 1:
                    drain_stores(c - 1)
                gathers.append(issue_gather(c + 1))
        drain_stores(NCHUNK - 2)
        drain_stores(NCHUNK - 1)

    return k(x2d, jnp.transpose(idx_t).reshape(B * NK_PAD))


def kernel(x):
    mask, idx_t = _select()
    out_kbd = _gather_sc(idx_t, x.reshape(B * S, D))
    return jnp.swapaxes(out_kbd, 0, 1), mask
